# baseline (speedup 1.0000x reference)
import dataclasses
import functools

import jax
import jax.numpy as jnp
from jax import lax
from jax.experimental import pallas as pl
from jax.experimental.pallas import tpu as pltpu
from jax.experimental.pallas import tpu_sc as plsc

E = 320000
D = 128
N = 10000
NW = 32
EPW = E // NW
S = 2000
NS = EPW // S
C = 80
NCHUNK = S // C
L = 16
G = C // L
FW = D // L


def _edge_dot_kernel(h_hbm, src_hbm, dst_hbm, out_hbm,
                     src_v, dst_v, u_a, v_a, u_b, v_b, ob_a, ob_b, tbuf, tbuf2,
                     sem_a, sem_b, sem_oa, sem_ob):
    cid = lax.axis_index("c")
    sid = lax.axis_index("s")
    wid = sid * 2 + cid
    base = wid * EPW

    lane = lax.iota(jnp.int32, L)

    def issue(ci, ub, vb, sem):
        off = ci * C
        pltpu.async_copy(h_hbm.at[src_v.at[pl.ds(off, C)]], ub, sem)
        pltpu.async_copy(h_hbm.at[dst_v.at[pl.ds(off, C)]], vb, sem)

    def drain(ci, ub, vb, sem):
        off = ci * C
        pltpu.make_async_copy(h_hbm.at[src_v.at[pl.ds(off, C)]], ub, sem).wait()
        pltpu.make_async_copy(h_hbm.at[dst_v.at[pl.ds(off, C)]], vb, sem).wait()

    def compute(ci, ub, vb, ob):
        def load_edge(eidx):
            us = [ub[eidx, pl.ds(f * L, L)] for f in range(FW)]
            vs = [vb[eidx, pl.ds(f * L, L)] for f in range(FW)]
            return us, vs

        def arith(tb, e, regs):
            us, vs = regs
            ps = [us[f] * vs[f] for f in range(FW)]
            acc = (((ps[0] + ps[1]) + (ps[2] + ps[3]))
                   + ((ps[4] + ps[5]) + (ps[6] + ps[7])))
            plsc.store_scatter(tb, [lane, lax.broadcast(e, (L,))], acc)

        def reduce_group(tb, g):
            tot = tb[0, pl.ds(0, L)]
            t1 = tb[1, pl.ds(0, L)] + tb[2, pl.ds(0, L)]
            t2 = tb[3, pl.ds(0, L)] + tb[4, pl.ds(0, L)]
            t3 = tb[5, pl.ds(0, L)] + tb[6, pl.ds(0, L)]
            t4 = tb[7, pl.ds(0, L)] + tb[8, pl.ds(0, L)]
            t5 = tb[9, pl.ds(0, L)] + tb[10, pl.ds(0, L)]
            t6 = tb[11, pl.ds(0, L)] + tb[12, pl.ds(0, L)]
            t7 = tb[13, pl.ds(0, L)] + tb[14, pl.ds(0, L)]
            t8 = tb[15, pl.ds(0, L)]
            tot = (((tot + t1) + (t2 + t3))
                   + ((t4 + t5) + (t6 + t7)) + t8)
            ob[pl.ds(g * L, L)] = tot

        tbufs = (tbuf, tbuf2)
        regs = load_edge(0)
        for g in range(G):
            tb = tbufs[g % 2]
            for e in range(L):
                nxt = g * L + e + 1
                regs_next = load_edge(nxt) if nxt < C else None
                arith(tb, e, regs)
                regs = regs_next
                if e == 1 and g >= 1:
                    reduce_group(tbufs[(g - 1) % 2], g - 1)
        reduce_group(tbufs[(G - 1) % 2], G - 1)

    @pl.loop(0, NS)
    def _super(sc):
        sbase = base + sc * S
        pltpu.sync_copy(src_hbm.at[pl.ds(sbase, S)], src_v)
        pltpu.sync_copy(dst_hbm.at[pl.ds(sbase, S)], dst_v)

        def issue_out(ci, ob, sem):
            pltpu.async_copy(ob, out_hbm.at[pl.ds(sbase + ci * C, C)], sem)

        def drain_out(ob, sem):
            pltpu.make_async_copy(ob, out_hbm.at[pl.ds(sbase, C)], sem).wait()

        issue(0, u_a, v_a, sem_a)

        @pl.loop(0, NCHUNK - 1, step=2)
        def _pair(ci):
            issue(ci + 1, u_b, v_b, sem_b)
            drain(ci, u_a, v_a, sem_a)

            @pl.when(ci >= 2)
            def _do():
                drain_out(ob_a, sem_oa)
                drain_out(ob_b, sem_ob)

            compute(ci, u_a, v_a, ob_a)
            issue_out(ci, ob_a, sem_oa)
            issue(ci + 2, u_a, v_a, sem_a)
            drain(ci + 1, u_b, v_b, sem_b)
            compute(ci + 1, u_b, v_b, ob_b)
            issue_out(ci + 1, ob_b, sem_ob)

        drain(NCHUNK - 1, u_a, v_a, sem_a)
        drain_out(ob_a, sem_oa)
        compute(NCHUNK - 1, u_a, v_a, ob_a)
        issue_out(NCHUNK - 1, ob_a, sem_oa)
        drain_out(ob_a, sem_oa)
        drain_out(ob_b, sem_ob)


@jax.jit
def kernel(h, edge_index):
    edge_index = edge_index.astype(jnp.int32)
    src = edge_index[0]
    dst = edge_index[1]
    mesh = plsc.VectorSubcoreMesh(core_axis_name="c", subcore_axis_name="s")
    cp = pltpu.CompilerParams()
    if "needs_layout_passes" in pltpu.CompilerParams.__dataclass_fields__:
        cp = dataclasses.replace(cp, needs_layout_passes=False)
    k = pl.kernel(
        _edge_dot_kernel,
        out_type=jax.ShapeDtypeStruct((E,), jnp.float32),
        mesh=mesh,
        scratch_types=[
            pltpu.VMEM((S,), jnp.int32),
            pltpu.VMEM((S,), jnp.int32),
            pltpu.VMEM((C, D), jnp.float32),
            pltpu.VMEM((C, D), jnp.float32),
            pltpu.VMEM((C, D), jnp.float32),
            pltpu.VMEM((C, D), jnp.float32),
            pltpu.VMEM((C,), jnp.float32),
            pltpu.VMEM((C,), jnp.float32),
            pltpu.VMEM((L, L + 1), jnp.float32),
            pltpu.VMEM((L, L + 1), jnp.float32),
            pltpu.SemaphoreType.DMA,
            pltpu.SemaphoreType.DMA,
            pltpu.SemaphoreType.DMA,
            pltpu.SemaphoreType.DMA,
        ],
        compiler_params=cp,
    )
    score = k(h, src, dst)
    return score.reshape(E, 1)


# speedup vs baseline: 1.2045x; 1.2045x over previous
import dataclasses
import functools

import jax
import jax.numpy as jnp
from jax import lax
from jax.experimental import pallas as pl
from jax.experimental.pallas import tpu as pltpu
from jax.experimental.pallas import tpu_sc as plsc

E = 320000
D = 128
DW = D // 2
N = 10000
NW = 32
EPW = E // NW
S = 2000
NS = EPW // S
C = 80
NCHUNK = S // C
L = 16
G = C // L
FW = DW // L


def _edge_dot_kernel(h_hbm, src_hbm, dst_hbm, out_hbm,
                     src_v, dst_v, u_a, v_a, u_b, v_b, ob_a, ob_b, tbuf, tbuf2,
                     h_sp, sem_a, sem_b, sem_oa, sem_ob):
    cid = lax.axis_index("c")
    sid = lax.axis_index("s")
    wid = sid * 2 + cid
    base = wid * EPW

    rows_per_sub = 624
    pltpu.sync_copy(h_hbm.at[pl.ds(sid * rows_per_sub, rows_per_sub)],
                    h_sp.at[pl.ds(sid * rows_per_sub, rows_per_sub)])

    @pl.when(sid == 0)
    def _tail():
        pltpu.sync_copy(
            h_hbm.at[pl.ds(16 * rows_per_sub, N - 16 * rows_per_sub)],
            h_sp.at[pl.ds(16 * rows_per_sub, N - 16 * rows_per_sub)])

    plsc.subcore_barrier()

    lane = lax.iota(jnp.int32, L)
    sh16 = lax.broadcast(jnp.int32(16), (L,))

    def issue(ci, ub, vb, sem):
        off = ci * C
        pltpu.async_copy(h_sp.at[src_v.at[pl.ds(off, C)]], ub, sem)
        pltpu.async_copy(h_sp.at[dst_v.at[pl.ds(off, C)]], vb, sem)

    def drain(ci, ub, vb, sem):
        off = ci * C
        pltpu.make_async_copy(h_sp.at[src_v.at[pl.ds(off, C)]], ub, sem).wait()
        pltpu.make_async_copy(h_sp.at[dst_v.at[pl.ds(off, C)]], vb, sem).wait()

    def compute(ci, ub, vb, ob):
        def load_edge(eidx):
            us = [ub[eidx, pl.ds(f * L, L)] for f in range(FW)]
            vs = [vb[eidx, pl.ds(f * L, L)] for f in range(FW)]
            return us, vs

        def arith(tb, e, regs):
            us, vs = regs
            ps = []
            for f in range(FW):
                u_hi = plsc.bitcast(us[f], jnp.float32)
                v_hi = plsc.bitcast(vs[f], jnp.float32)
                u_lo = plsc.bitcast(lax.shift_left(us[f], sh16),
                                    jnp.float32)
                v_lo = plsc.bitcast(lax.shift_left(vs[f], sh16),
                                    jnp.float32)
                ps.append(u_hi * v_hi)
                ps.append(u_lo * v_lo)
            acc = (((ps[0] + ps[1]) + (ps[2] + ps[3]))
                   + ((ps[4] + ps[5]) + (ps[6] + ps[7])))
            plsc.store_scatter(tb, [lane, lax.broadcast(e, (L,))], acc)

        def reduce_group(tb, g):
            tot = tb[0, pl.ds(0, L)]
            t1 = tb[1, pl.ds(0, L)] + tb[2, pl.ds(0, L)]
            t2 = tb[3, pl.ds(0, L)] + tb[4, pl.ds(0, L)]
            t3 = tb[5, pl.ds(0, L)] + tb[6, pl.ds(0, L)]
            t4 = tb[7, pl.ds(0, L)] + tb[8, pl.ds(0, L)]
            t5 = tb[9, pl.ds(0, L)] + tb[10, pl.ds(0, L)]
            t6 = tb[11, pl.ds(0, L)] + tb[12, pl.ds(0, L)]
            t7 = tb[13, pl.ds(0, L)] + tb[14, pl.ds(0, L)]
            t8 = tb[15, pl.ds(0, L)]
            tot = (((tot + t1) + (t2 + t3))
                   + ((t4 + t5) + (t6 + t7)) + t8)
            ob[pl.ds(g * L, L)] = tot

        tbufs = (tbuf, tbuf2)
        regs = load_edge(0)
        for g in range(G):
            tb = tbufs[g % 2]
            for e in range(L):
                nxt = g * L + e + 1
                regs_next = load_edge(nxt) if nxt < C else None
                arith(tb, e, regs)
                regs = regs_next
                if e == 1 and g >= 1:
                    reduce_group(tbufs[(g - 1) % 2], g - 1)
        reduce_group(tbufs[(G - 1) % 2], G - 1)

    @pl.loop(0, NS)
    def _super(sc):
        sbase = base + sc * S
        pltpu.sync_copy(src_hbm.at[pl.ds(sbase, S)], src_v)
        pltpu.sync_copy(dst_hbm.at[pl.ds(sbase, S)], dst_v)

        def issue_out(ci, ob, sem):
            pltpu.async_copy(ob, out_hbm.at[pl.ds(sbase + ci * C, C)], sem)

        def drain_out(ob, sem):
            pltpu.make_async_copy(ob, out_hbm.at[pl.ds(sbase, C)], sem).wait()

        issue(0, u_a, v_a, sem_a)

        @pl.loop(0, NCHUNK - 1, step=2)
        def _pair(ci):
            issue(ci + 1, u_b, v_b, sem_b)
            drain(ci, u_a, v_a, sem_a)

            @pl.when(ci >= 2)
            def _do():
                drain_out(ob_a, sem_oa)
                drain_out(ob_b, sem_ob)

            compute(ci, u_a, v_a, ob_a)
            issue_out(ci, ob_a, sem_oa)
            issue(ci + 2, u_a, v_a, sem_a)
            drain(ci + 1, u_b, v_b, sem_b)
            compute(ci + 1, u_b, v_b, ob_b)
            issue_out(ci + 1, ob_b, sem_ob)

        drain(NCHUNK - 1, u_a, v_a, sem_a)
        drain_out(ob_a, sem_oa)
        compute(NCHUNK - 1, u_a, v_a, ob_a)
        issue_out(NCHUNK - 1, ob_a, sem_oa)
        drain_out(ob_a, sem_oa)
        drain_out(ob_b, sem_ob)


@jax.jit
def kernel(h, edge_index):
    edge_index = edge_index.astype(jnp.int32)
    src = edge_index[0]
    dst = edge_index[1]
    h_packed = jax.lax.bitcast_convert_type(
        h.astype(jnp.bfloat16).reshape(N, DW, 2), jnp.int32)

    mesh = plsc.VectorSubcoreMesh(core_axis_name="c", subcore_axis_name="s")
    cp = pltpu.CompilerParams()
    if "needs_layout_passes" in pltpu.CompilerParams.__dataclass_fields__:
        cp = dataclasses.replace(cp, needs_layout_passes=False)
    k = pl.kernel(
        _edge_dot_kernel,
        out_type=jax.ShapeDtypeStruct((E,), jnp.float32),
        mesh=mesh,
        scratch_types=[
            pltpu.VMEM((S,), jnp.int32),
            pltpu.VMEM((S,), jnp.int32),
            pltpu.VMEM((C, DW), jnp.int32),
            pltpu.VMEM((C, DW), jnp.int32),
            pltpu.VMEM((C, DW), jnp.int32),
            pltpu.VMEM((C, DW), jnp.int32),
            pltpu.VMEM((C,), jnp.float32),
            pltpu.VMEM((C,), jnp.float32),
            pltpu.VMEM((L, L + 1), jnp.float32),
            pltpu.VMEM((L, L + 1), jnp.float32),
            pltpu.VMEM_SHARED((N, DW), jnp.int32),
            pltpu.SemaphoreType.DMA,
            pltpu.SemaphoreType.DMA,
            pltpu.SemaphoreType.DMA,
            pltpu.SemaphoreType.DMA,
        ],
        compiler_params=cp,
    )
    score = k(h_packed, src, dst)
    return score.reshape(E, 1)
